# Initial kernel scaffold; baseline (speedup 1.0000x reference)
import jax
import jax.numpy as jnp
from jax.experimental import pallas as pl


def kernel(edge_index, z, z_table, W0, b0, W1, b1, W2, b2, W3, b3, conv1_w, conv1_b, conv2_w, conv2_b, lin1_w, lin1_b, lin2_w, lin2_b):
    raise NotImplementedError("write your pallas kernel here")


# speedup vs baseline: 6.4058x; 6.4058x over previous
import functools

import jax
import jax.numpy as jnp
from jax import lax
from jax.experimental import pallas as pl
from jax.experimental.pallas import tpu as pltpu
from jax.experimental.pallas import tpu_sc as plsc

N = 10000
NP = 10240
E = 320000
H = 128
NT = 16
NC = 2
NW = NC * NT
EPW = E // NW
ECH = 80
ENCH = EPW // ECH
STRIPE = NP // NT
XCH = 80
XNCH = N // XCH

_mesh = plsc.VectorSubcoreMesh(core_axis_name="c", subcore_axis_name="s")
_f32 = jnp.float32
_sc_params = pltpu.CompilerParams(needs_layout_passes=False)


def _zero_1d(ref, n):
    z = jnp.zeros((16,), _f32)

    def body(i, _):
        ref[pl.ds(i * 16, 16)] = z
        return 0

    lax.fori_loop(0, n // 16, body, 0)


@functools.partial(
    pl.kernel,
    out_type=[
        jax.ShapeDtypeStruct((NC, NP), _f32),
        jax.ShapeDtypeStruct((NC, NP), _f32),
        jax.ShapeDtypeStruct((N, H), _f32),
    ],
    mesh=_mesh,
    scratch_types=[
        pltpu.VMEM_SHARED((2, NT, NP), _f32),
        pltpu.VMEM((EPW,), jnp.int32),
        pltpu.VMEM((EPW,), jnp.int32),
        pltpu.VMEM((NP,), _f32),
        pltpu.VMEM((NP,), _f32),
        pltpu.VMEM((XCH,), jnp.int32),
        pltpu.VMEM((XCH, H), _f32),
        pltpu.VMEM((STRIPE,), _f32),
        pltpu.VMEM((STRIPE,), _f32),
        pltpu.SemaphoreType.DMA,
    ],
    compiler_params=_sc_params,
)
def _sc_prep(src_hbm, dst_hbm, z_hbm, zt_hbm, degs_hbm, degd_hbm, x0_hbm,
             parts_sh, src_v, dst_v, hs_v, hd_v, zi_v, rows_v, acc_v, tmp_v,
             sem):
    c = lax.axis_index("c")
    s = lax.axis_index("s")
    w = c * NT + s
    ones = jnp.full((16,), 1.0, _f32)

    _zero_1d(hs_v, NP)
    _zero_1d(hd_v, NP)
    pltpu.sync_copy(src_hbm.at[pl.ds(w * EPW, EPW)], src_v)
    pltpu.sync_copy(dst_hbm.at[pl.ds(w * EPW, EPW)], dst_v)

    def hist(i, _):
        s16 = src_v[pl.ds(i * 16, 16)]
        d16 = dst_v[pl.ds(i * 16, 16)]
        plsc.addupdate_scatter(hs_v, [s16], ones)
        plsc.addupdate_scatter(hd_v, [d16], ones)
        return 0

    lax.fori_loop(0, EPW // 16, hist, 0)
    pltpu.sync_copy(hs_v, parts_sh.at[0, s, :])
    pltpu.sync_copy(hd_v, parts_sh.at[1, s, :])
    plsc.subcore_barrier()

    for hidx, out in ((0, degs_hbm), (1, degd_hbm)):
        _zero_1d(acc_v, STRIPE)

        def comb(p, _):
            pltpu.sync_copy(parts_sh.at[hidx, p, pl.ds(s * STRIPE, STRIPE)],
                            tmp_v)

            def addv(j, _):
                acc_v[pl.ds(j * 16, 16)] = (
                    acc_v[pl.ds(j * 16, 16)] + tmp_v[pl.ds(j * 16, 16)])
                return 0

            lax.fori_loop(0, STRIPE // 16, addv, 0)
            return 0

        lax.fori_loop(0, NT, comb, 0)
        pltpu.sync_copy(acc_v, out.at[c, pl.ds(s * STRIPE, STRIPE)])

    for t in range(4):
        j = w + t * NW

        @pl.when(j < XNCH)
        def _():
            pltpu.sync_copy(z_hbm.at[pl.ds(j * XCH, XCH)], zi_v)
            pltpu.async_copy(zt_hbm.at[zi_v], rows_v, sem).wait()
            pltpu.sync_copy(rows_v, x0_hbm.at[pl.ds(j * XCH, XCH), :])


@functools.partial(
    pl.kernel,
    out_type=jax.ShapeDtypeStruct((NC, NP, H), _f32),
    mesh=_mesh,
    scratch_types=[
        pltpu.VMEM_SHARED((NP, H), _f32),
        pltpu.VMEM((16, H), _f32),
        pltpu.VMEM((ECH, H), _f32),
        pltpu.VMEM((ECH,), jnp.int32),
        pltpu.VMEM((ECH,), jnp.int32),
        pltpu.SemaphoreType.DMA,
    ],
    compiler_params=_sc_params,
)
def _sc_mp(h_hbm, src_hbm, dst_hbm, out_hbm, acc_sh, zb_v, rows_v, si_v,
           di_v, sem):
    c = lax.axis_index("c")
    s = lax.axis_index("s")
    w = c * NT + s
    z = jnp.zeros((16,), _f32)

    for r in range(16):
        for col in range(H // 16):
            zb_v[r, pl.ds(col * 16, 16)] = z

    def zacc(i, _):
        pltpu.sync_copy(zb_v, acc_sh.at[pl.ds(s * STRIPE + i * 16, 16), :])
        return 0

    lax.fori_loop(0, STRIPE // 16, zacc, 0)
    plsc.subcore_barrier()

    def step(i, _):
        off = w * EPW + i * ECH
        pltpu.sync_copy(src_hbm.at[pl.ds(off, ECH)], si_v)
        pltpu.sync_copy(dst_hbm.at[pl.ds(off, ECH)], di_v)
        pltpu.async_copy(h_hbm.at[si_v], rows_v, sem).wait()
        pltpu.sync_copy(rows_v, acc_sh.at[di_v], add=True)
        return 0

    lax.fori_loop(0, ENCH, step, 0)
    plsc.subcore_barrier()
    pltpu.sync_copy(acc_sh.at[pl.ds(s * STRIPE, STRIPE), :],
                    out_hbm.at[c, pl.ds(s * STRIPE, STRIPE), :])


@functools.partial(
    pl.kernel,
    out_type=jax.ShapeDtypeStruct((NC, NP), _f32),
    mesh=_mesh,
    scratch_types=[
        pltpu.VMEM_SHARED((NT, NP), _f32),
        pltpu.VMEM((N,), _f32),
        pltpu.VMEM((EPW,), jnp.int32),
        pltpu.VMEM((EPW,), jnp.int32),
        pltpu.VMEM((NP,), _f32),
        pltpu.VMEM((STRIPE,), _f32),
        pltpu.VMEM((STRIPE,), _f32),
    ],
    compiler_params=_sc_params,
)
def _sc_narrow(h3_hbm, src_hbm, dst_hbm, out_hbm, parts_sh, h3_v, src_v,
               dst_v, acch_v, acc_v, tmp_v):
    c = lax.axis_index("c")
    s = lax.axis_index("s")
    w = c * NT + s

    pltpu.sync_copy(h3_hbm, h3_v)
    pltpu.sync_copy(src_hbm.at[pl.ds(w * EPW, EPW)], src_v)
    pltpu.sync_copy(dst_hbm.at[pl.ds(w * EPW, EPW)], dst_v)
    _zero_1d(acch_v, NP)

    def seg(i, _):
        s16 = src_v[pl.ds(i * 16, 16)]
        d16 = dst_v[pl.ds(i * 16, 16)]
        vals = plsc.load_gather(h3_v, [s16])
        plsc.addupdate_scatter(acch_v, [d16], vals)
        return 0

    lax.fori_loop(0, EPW // 16, seg, 0)
    pltpu.sync_copy(acch_v, parts_sh.at[s, :])
    plsc.subcore_barrier()

    _zero_1d(acc_v, STRIPE)

    def comb(p, _):
        pltpu.sync_copy(parts_sh.at[p, pl.ds(s * STRIPE, STRIPE)], tmp_v)

        def addv(j, _):
            acc_v[pl.ds(j * 16, 16)] = (
                acc_v[pl.ds(j * 16, 16)] + tmp_v[pl.ds(j * 16, 16)])
            return 0

        lax.fori_loop(0, STRIPE // 16, addv, 0)
        return 0

    lax.fori_loop(0, NT, comb, 0)
    pltpu.sync_copy(acc_v, out_hbm.at[c, pl.ds(s * STRIPE, STRIPE)])


def _norm_body(degs_ref, degd_ref, dno_ref, dni_ref):
    so = degs_ref[0:1, :] + degs_ref[1:2, :]
    si = degd_ref[0:1, :] + degd_ref[1:2, :]
    dno_ref[:] = lax.rsqrt(jnp.maximum(so, 1.0))
    dni_ref[:] = lax.rsqrt(jnp.maximum(si, 1.0))


def _mm0_body(x0_ref, dno_ref, w_ref, h_ref):
    h_ref[:] = lax.dot_general(x0_ref[:] * dno_ref[:], w_ref[:],
                               (((1,), (0,)), ((), ())),
                               preferred_element_type=_f32)


def _layer_body(aggp_ref, dni_ref, dno_ref, b_ref, w_ref, x_ref, rm_ref,
                h_ref):
    agg = aggp_ref[0, :N, :] + aggp_ref[1, :N, :]
    x = jnp.tanh(agg * dni_ref[:] + b_ref[0:1, :])
    x_ref[:] = x
    rm_ref[:] = jnp.max(x, axis=1, keepdims=True)
    h_ref[:] = lax.dot_general(x * dno_ref[:], w_ref[:],
                               (((1,), (0,)), ((), ())),
                               preferred_element_type=_f32)


K = 30
TL = 3 * H + 1
SP = 512


def _head_body(aggn_ref, dni_ref, rm1_ref, rm2_ref, rm3_ref, x1_ref, x2_ref,
               x3_ref, b3_ref, c1w_ref, c1b_ref, c2w_ref, c2b_ref, lw1_ref,
               lb1_ref, lw2_ref, lb2_ref, out_ref, sel_ref, p4_ref):
    agg3 = aggn_ref[0:1, :N] + aggn_ref[1:2, :N]
    x4 = jnp.tanh(agg3 * dni_ref[0:1, :] + b3_ref[0:1, :])
    keys0 = jnp.maximum(jnp.maximum(rm1_ref[0:1, :], rm2_ref[0:1, :]),
                        jnp.maximum(rm3_ref[0:1, :], x4))
    iota = lax.broadcasted_iota(jnp.int32, (1, N), 1)

    def it(i, keys):
        m = jnp.max(keys)
        idx = jnp.min(jnp.where(keys == m, iota, N))
        oh = iota == idx
        sel_ref[pl.ds(i, 1), :] = oh.astype(_f32)
        p4_ref[pl.ds(i, 1), :] = jnp.sum(jnp.where(oh, x4, 0.0)).reshape(1, 1)
        return jnp.where(oh, -2.0, keys)

    lax.fori_loop(0, K, it, keys0)
    sel = sel_ref[:]
    dn = (((1,), (0,)), ((), ()))
    p1 = lax.dot_general(sel, x1_ref[:], dn, preferred_element_type=_f32)
    p2 = lax.dot_general(sel, x2_ref[:], dn, preferred_element_type=_f32)
    p3 = lax.dot_general(sel, x3_ref[:], dn, preferred_element_type=_f32)
    pooled = jnp.concatenate(
        [p1, p2, p3, p4_ref[:], jnp.full((K, SP - TL), 2.0, _f32)], axis=1)

    x = pooled
    lane = lax.broadcasted_iota(jnp.int32, (1, SP), 1)
    k = 2
    while k <= SP:
        j = k // 2
        while j >= 1:
            lower = (lane & j) == 0
            asc = (lane & k) == 0
            partner = jnp.where(lower, pltpu.roll(x, SP - j, 1),
                                pltpu.roll(x, j, 1))
            want_min = asc == lower
            x = jnp.where(want_min, jnp.minimum(x, partner),
                          jnp.maximum(x, partner))
            j //= 2
        k *= 2
    sfeat = x[:, :TL]

    dc = (((1,), (1,)), ((), ()))
    t1 = jnp.maximum(
        lax.dot_general(sfeat, c1w_ref[:], dc, preferred_element_type=_f32)
        + c1b_ref[0:1, :], 0.0)
    t2 = jnp.max(t1.reshape(K // 2, 2, 16), axis=1)
    a = jnp.zeros((11, 32), _f32)
    for o in range(5):
        a = a + lax.dot_general(t2[o:o + 11, :], c2w_ref[o], dc,
                                preferred_element_type=_f32)
    a = jnp.maximum(a + c2b_ref[0:1, :], 0.0)
    hl = jnp.zeros((1, 128), _f32)
    for l in range(11):
        hl = hl + lax.dot_general(a[l:l + 1, :], lw1_ref[l], dc,
                                  preferred_element_type=_f32)
    hl = jnp.maximum(hl + lb1_ref[0:1, :], 0.0)
    out_ref[:] = (jnp.sum(hl * lw2_ref[0:1, :])
                  + jnp.sum(lb2_ref[0:1, :])).reshape(1, 1)


def _tc_norm(degs, degd):
    return pl.pallas_call(
        _norm_body,
        out_shape=[jax.ShapeDtypeStruct((1, NP), _f32),
                   jax.ShapeDtypeStruct((1, NP), _f32)],
    )(degs, degd)


def _tc_mm0(x0, dno_col, w0):
    return pl.pallas_call(
        _mm0_body,
        out_shape=jax.ShapeDtypeStruct((N, H), _f32),
    )(x0, dno_col, w0)


def _tc_layer(aggp, dni_col, dno_col, b_row, w):
    dout = w.shape[1]
    return pl.pallas_call(
        _layer_body,
        out_shape=[jax.ShapeDtypeStruct((N, H), _f32),
                   jax.ShapeDtypeStruct((N, 1), _f32),
                   jax.ShapeDtypeStruct((N, dout), _f32)],
    )(aggp, dni_col, dno_col, b_row, w)


def _tc_head(aggn, dni_row, rm1, rm2, rm3, x1, x2, x3, b3, c1w, c1b, c2w,
             c2b, lw1, lb1, lw2, lb2):
    return pl.pallas_call(
        _head_body,
        out_shape=jax.ShapeDtypeStruct((1, 1), _f32),
        scratch_shapes=[pltpu.VMEM((K, N), _f32), pltpu.VMEM((K, 1), _f32)],
    )(aggn, dni_row, rm1, rm2, rm3, x1, x2, x3, b3, c1w, c1b, c2w, c2b, lw1,
      lb1, lw2, lb2)


def _pad8(x):
    return jnp.pad(x, ((0, 8 - x.shape[0]), (0, 0)))


def kernel(edge_index, z, z_table, W0, b0, W1, b1, W2, b2, W3, b3, conv1_w,
           conv1_b, conv2_w, conv2_b, lin1_w, lin1_b, lin2_w, lin2_b):
    src = edge_index[0]
    dst = edge_index[1]
    degs, degd, x0 = _sc_prep(src, dst, z, z_table)
    dno_row, dni_row = _tc_norm(degs, degd)
    dno_col = dno_row[0, :N].reshape(N, 1)
    dni_col = dni_row[0, :N].reshape(N, 1)
    h0 = _tc_mm0(x0, dno_col, W0)
    agg0 = _sc_mp(h0, src, dst)
    x1, rm1, h1 = _tc_layer(agg0, dni_col, dno_col, _pad8(b0.reshape(1, H)),
                            W1)
    agg1 = _sc_mp(h1, src, dst)
    x2, rm2, h2 = _tc_layer(agg1, dni_col, dno_col, _pad8(b1.reshape(1, H)),
                            W2)
    agg2 = _sc_mp(h2, src, dst)
    x3, rm3, h3 = _tc_layer(agg2, dni_col, dno_col, _pad8(b2.reshape(1, H)),
                            W3)
    aggn = _sc_narrow(h3.reshape(N), src, dst)
    return _tc_head(
        aggn, _pad8(dni_row[:, :N]), _pad8(rm1.reshape(1, N)),
        _pad8(rm2.reshape(1, N)), _pad8(rm3.reshape(1, N)), x1, x2, x3,
        _pad8(jnp.broadcast_to(b3.reshape(1, 1), (1, N))), conv1_w[:, 0, :],
        _pad8(conv1_b.reshape(1, 16)),
        jnp.stack([conv2_w[:, :, o] for o in range(5)], axis=0),
        _pad8(conv2_b.reshape(1, 32)),
        jnp.stack([lin1_w.reshape(128, 32, 11)[:, :, l] for l in range(11)],
                  axis=0),
        _pad8(lin1_b.reshape(1, 128)), _pad8(lin2_w),
        jnp.pad(lin2_b.reshape(1, 1), ((0, 7), (0, 127))))
